# baseline (device time: 334612 ns/iter reference)
import numpy as np

import jax
import jax.numpy as jnp
from jax import lax
from jax.experimental import pallas as pl
from jax.experimental.pallas import tpu as pltpu

N_DEV = 8
SQ = 2048
SKV = 2048
D_MODEL = 1024
DH = 128
HEADS_PER_SHARD = 8
NBLK = SQ // 64
CHUNK = SQ // N_DEV
SEG = 4
SEGR = CHUNK // SEG
SCALE = 0.08838834764831843
NEG = -1e9
BF = jnp.bfloat16

_blocks = np.arange(NBLK)
_ORDER = np.concatenate(
    [_blocks[_blocks % 3 == 0], _blocks[_blocks % 3 == 1],
     _blocks[_blocks % 3 == 2]]
).tolist()
_n0 = (NBLK + 2) // 3 * 64
_n1 = (NBLK + 1) // 3 * 64
_n2 = NBLK // 3 * 64
REG = {0: (0, _n0), 1: (_n0, _n1), 2: (_n0 + _n1, _n2)}
CLS = [
    (REG[0], REG[0], None),
    (REG[1], REG[2], REG[1]),
    (REG[2], REG[1], REG[2]),
]

_CYCLES = []
_seen = [False] * NBLK
for _i in range(NBLK):
    if _seen[_i]:
        continue
    _cyc, _j = [], _i
    while not _seen[_j]:
        _seen[_j] = True
        _cyc.append(_j)
        _j = _ORDER[_j]
    if len(_cyc) > 1:
        _CYCLES.append(_cyc)


def _body(x_ref, wq_ref, k_hbm, v_hbm, wo_ref, out_ref,
          k_buf, v_buf, qp, rs_recv, ag_recv, rs_sb, ag_sb, blk_tmp,
          kv_sems, rs_ssem, rs_rsem, ag_ssem, ag_rsem):
    my = lax.axis_index("i")
    right = (my + 1) % N_DEV
    left = (my + N_DEV - 1) % N_DEV

    barrier_sem = pltpu.get_barrier_semaphore()
    for nbr in (left, right):
        pl.semaphore_signal(barrier_sem, inc=1, device_id=(nbr,),
                            device_id_type=pl.DeviceIdType.MESH)
    pl.semaphore_wait(barrier_sem, 2)

    def stage(j, slot):
        col = (my * HEADS_PER_SHARD + j) * DH
        descs = []
        for i, ob in enumerate(_ORDER):
            for hbm, buf, si in ((k_hbm, k_buf, 0), (v_hbm, v_buf, 1)):
                d = pltpu.make_async_copy(
                    hbm.at[pl.ds(64 * ob, 64), pl.ds(col, DH)],
                    buf.at[slot, pl.ds(64 * i, 64), :],
                    kv_sems.at[si, slot],
                )
                d.start()
                descs.append(d)
        return descs

    descs = stage(0, 0)
    for j in range(HEADS_PER_SHARD):
        slot = j % 2
        for d in descs:
            d.wait()
        if j < HEADS_PER_SHARD - 1:
            descs = stage(j + 1, 1 - slot)
        c0, c1 = j * DH, (j + 1) * DH
        wq_j = wq_ref[:, c0:c1].astype(BF)
        wo_j = wo_ref[c0:c1, :].astype(BF)
        qf = jnp.dot(x_ref[:, :].astype(BF), wq_j,
                     preferred_element_type=jnp.float32).astype(BF)
        for i, ob in enumerate(_ORDER):
            qp[64 * i:64 * i + 64, :] = qf[64 * ob:64 * ob + 64, :]
        for (q0, qn), (a0, an), breg in CLS:
            qh = qp[q0:q0 + qn, :]
            k_a = k_buf[slot, a0:a0 + an, :].astype(BF)
            v_a = v_buf[slot, a0:a0 + an, :].astype(BF)
            s_a = lax.dot_general(
                qh, k_a, (((1,), (1,)), ((), ())),
                preferred_element_type=jnp.float32) * SCALE
            m_a = jnp.max(s_a, axis=-1, keepdims=True)
            if breg is None:
                w = jnp.exp(s_a - m_a)
                ctx = jnp.dot(w.astype(BF), v_a,
                              preferred_element_type=jnp.float32)
                ctx = ctx / jnp.sum(w, axis=-1, keepdims=True)
            else:
                b0, bn = breg
                s_b = lax.dot_general(
                    qh, k_buf[slot, b0:b0 + bn, :].astype(BF),
                    (((1,), (1,)), ((), ())),
                    preferred_element_type=jnp.float32) * SCALE
                rb = lax.broadcasted_iota(jnp.int32, (qn, 1), 0) // 64
                cb = lax.broadcasted_iota(jnp.int32, (1, bn), 1) // 64
                s_b = jnp.where(rb == cb, s_b, NEG)
                s_c = lax.dot_general(
                    qh, k_buf[slot, 0:64, :].astype(BF),
                    (((1,), (1,)), ((), ())),
                    preferred_element_type=jnp.float32) * SCALE
                m = jnp.maximum(
                    m_a,
                    jnp.maximum(jnp.max(s_b, axis=-1, keepdims=True),
                                jnp.max(s_c, axis=-1, keepdims=True)))
                e_a = jnp.exp(s_a - m)
                e_b = jnp.exp(s_b - m)
                e_c = jnp.exp(s_c - m)
                denom = (jnp.sum(e_a, axis=-1, keepdims=True)
                         + jnp.sum(e_b, axis=-1, keepdims=True)
                         + jnp.sum(e_c, axis=-1, keepdims=True))
                ctx = (jnp.dot(e_a.astype(BF), v_a,
                               preferred_element_type=jnp.float32)
                       + jnp.dot(e_b.astype(BF),
                                 v_buf[slot, b0:b0 + bn, :].astype(BF),
                                 preferred_element_type=jnp.float32)
                       + jnp.dot(e_c.astype(BF),
                                 v_buf[slot, 0:64, :].astype(BF),
                                 preferred_element_type=jnp.float32))
                ctx = ctx / denom
            part = jnp.dot(ctx.astype(BF), wo_j,
                           preferred_element_type=jnp.float32)
            if j == 0:
                out_ref[q0:q0 + qn, :] = part
            else:
                out_ref[q0:q0 + qn, :] = out_ref[q0:q0 + qn, :] + part

    def rs_desc(h, s):
        return pltpu.make_async_remote_copy(
            src_ref=rs_sb.at[h, pl.ds(s * SEGR, SEGR), :],
            dst_ref=rs_recv.at[h, pl.ds(s * SEGR, SEGR), :],
            send_sem=rs_ssem.at[h, s],
            recv_sem=rs_rsem.at[h, s],
            device_id=(right,),
            device_id_type=pl.DeviceIdType.MESH,
        )

    rs_sends = []
    rs_sb[0] = out_ref[pl.ds(my * CHUNK, CHUNK), :].astype(BF)
    for s in range(SEG):
        d = rs_desc(0, s)
        d.start()
        rs_sends.append(d)
    for h in range(N_DEV - 1):
        recv_c = (my - h - 1) % N_DEV
        for s in range(SEG):
            rs_desc(h, s).wait_recv()
            rows = pl.ds(recv_c * CHUNK + s * SEGR, SEGR)
            acc = (out_ref[rows, :]
                   + rs_recv[h, s * SEGR:(s + 1) * SEGR, :].astype(
                       jnp.float32))
            out_ref[rows, :] = acc
            if h < N_DEV - 2:
                rs_sb[h + 1, s * SEGR:(s + 1) * SEGR, :] = acc.astype(BF)
                d = rs_desc(h + 1, s)
                d.start()
                rs_sends.append(d)
    for d in rs_sends:
        d.wait_send()

    def ag_desc(h, s):
        if h == 0:
            src = ag_sb.at[pl.ds(s * SEGR, SEGR), :]
        else:
            src = ag_recv.at[h - 1, pl.ds(s * SEGR, SEGR), :]
        return pltpu.make_async_remote_copy(
            src_ref=src,
            dst_ref=ag_recv.at[h, pl.ds(s * SEGR, SEGR), :],
            send_sem=ag_ssem.at[h, s],
            recv_sem=ag_rsem.at[h, s],
            device_id=(right,),
            device_id_type=pl.DeviceIdType.MESH,
        )

    ag_sends = []
    ag_sb[...] = out_ref[
        pl.ds(((my + 1) % N_DEV) * CHUNK, CHUNK), :].astype(BF)
    for s in range(SEG):
        d = ag_desc(0, s)
        d.start()
        ag_sends.append(d)
    for h in range(N_DEV - 1):
        for s in range(SEG):
            ag_desc(h, s).wait_recv()
            if h < N_DEV - 2:
                d = ag_desc(h + 1, s)
                d.start()
                ag_sends.append(d)
            out_ref[pl.ds(((my - h) % N_DEV) * CHUNK + s * SEGR, SEGR), :] = (
                ag_recv[h, s * SEGR:(s + 1) * SEGR, :].astype(jnp.float32))
    for d in ag_sends:
        d.wait_send()

    for cyc in _CYCLES:
        k = len(cyc)
        blk_tmp[...] = out_ref[64 * cyc[k - 1]:64 * cyc[k - 1] + 64, :]
        for m in range(k - 1, 0, -1):
            out_ref[64 * cyc[m]:64 * cyc[m] + 64, :] = (
                out_ref[64 * cyc[m - 1]:64 * cyc[m - 1] + 64, :])
        out_ref[64 * cyc[0]:64 * cyc[0] + 64, :] = blk_tmp[...]


def kernel(x, Wq, K_ext, V_ext, Wo):
    x2 = x.reshape(SQ, D_MODEL)
    K2 = K_ext.reshape(SKV, 64 * DH)
    V2 = V_ext.reshape(SKV, 64 * DH)

    out = pl.pallas_call(
        _body,
        out_shape=jax.ShapeDtypeStruct((SQ, D_MODEL), jnp.float32),
        in_specs=[
            pl.BlockSpec(memory_space=pltpu.VMEM),
            pl.BlockSpec(memory_space=pltpu.VMEM),
            pl.BlockSpec(memory_space=pltpu.MemorySpace.HBM),
            pl.BlockSpec(memory_space=pltpu.MemorySpace.HBM),
            pl.BlockSpec(memory_space=pltpu.VMEM),
        ],
        out_specs=pl.BlockSpec(memory_space=pltpu.VMEM),
        scratch_shapes=[
            pltpu.VMEM((2, SKV, DH), jnp.float32),
            pltpu.VMEM((2, SKV, DH), jnp.float32),
            pltpu.VMEM((SQ, DH), BF),
            pltpu.VMEM((N_DEV - 1, CHUNK, D_MODEL), BF),
            pltpu.VMEM((N_DEV - 1, CHUNK, D_MODEL), BF),
            pltpu.VMEM((N_DEV - 1, CHUNK, D_MODEL), BF),
            pltpu.VMEM((CHUNK, D_MODEL), BF),
            pltpu.VMEM((64, D_MODEL), jnp.float32),
            pltpu.SemaphoreType.DMA((2, 2)),
            pltpu.SemaphoreType.DMA((N_DEV - 1, SEG)),
            pltpu.SemaphoreType.DMA((N_DEV - 1, SEG)),
            pltpu.SemaphoreType.DMA((N_DEV - 1, SEG)),
            pltpu.SemaphoreType.DMA((N_DEV - 1, SEG)),
        ],
        compiler_params=pltpu.CompilerParams(
            collective_id=0,
            vmem_limit_bytes=63 * 1024 * 1024,
        ),
    )(x2, Wq, K2, V2, Wo)
    return out.reshape(1, SQ, D_MODEL)


# device time: 183886 ns/iter; 1.8197x vs baseline; 1.8197x over previous
import numpy as np

import jax
import jax.numpy as jnp
from jax import lax
from jax.experimental import pallas as pl
from jax.experimental.pallas import tpu as pltpu

N_DEV = 8
SQ = 2048
SKV = 2048
D_MODEL = 1024
DH = 128
HEADS_PER_SHARD = 8
NBLK = SQ // 64
CHUNK = SQ // N_DEV
SEG = 4
SEGR = CHUNK // SEG
SCALE = 0.08838834764831843
NEG = -1e9
BF = jnp.bfloat16

_blocks = np.arange(NBLK)
_ORDER = np.concatenate(
    [_blocks[_blocks % 3 == 0], _blocks[_blocks % 3 == 1],
     _blocks[_blocks % 3 == 2]]
).tolist()
_n0 = (NBLK + 2) // 3 * 64
_n1 = (NBLK + 1) // 3 * 64
_n2 = NBLK // 3 * 64
REG = {0: (0, _n0), 1: (_n0, _n1), 2: (_n0 + _n1, _n2)}
CLS = [
    (REG[0], REG[0], None),
    (REG[1], REG[2], REG[1]),
    (REG[2], REG[1], REG[2]),
]

_CYCLES = []
_seen = [False] * NBLK
for _i in range(NBLK):
    if _seen[_i]:
        continue
    _cyc, _j = [], _i
    while not _seen[_j]:
        _seen[_j] = True
        _cyc.append(_j)
        _j = _ORDER[_j]
    if len(_cyc) > 1:
        _CYCLES.append(_cyc)


def _body(x_ref, wq_ref, k_hbm, v_hbm, wo_ref, out_ref,
          k_buf, v_buf, qp, rs_recv, ag_recv, rs_sb, ag_sb, blk_tmp,
          kv_sems, rs_ssem, rs_rsem, ag_ssem, ag_rsem):
    my = lax.axis_index("i")
    right = (my + 1) % N_DEV
    left = (my + N_DEV - 1) % N_DEV

    barrier_sem = pltpu.get_barrier_semaphore()
    for nbr in (left, right):
        pl.semaphore_signal(barrier_sem, inc=1, device_id=(nbr,),
                            device_id_type=pl.DeviceIdType.MESH)
    pl.semaphore_wait(barrier_sem, 2)

    def stage(j, slot):
        hidx = my * HEADS_PER_SHARD + j
        descs = []
        for i, ob in enumerate(_ORDER):
            for hbm, buf, si in ((k_hbm, k_buf, 0), (v_hbm, v_buf, 1)):
                d = pltpu.make_async_copy(
                    hbm.at[0, pl.ds(64 * ob, 64), hidx, :],
                    buf.at[slot, pl.ds(64 * i, 64), :],
                    kv_sems.at[si, slot],
                )
                d.start()
                descs.append(d)
        return descs

    descs = stage(0, 0)
    for j in range(HEADS_PER_SHARD):
        slot = j % 2
        for d in descs:
            d.wait()
        if j < HEADS_PER_SHARD - 1:
            descs = stage(j + 1, 1 - slot)
        c0, c1 = j * DH, (j + 1) * DH
        wq_j = wq_ref[:, c0:c1].astype(BF)
        wo_j = wo_ref[c0:c1, :].astype(BF)
        qf = jnp.dot(x_ref[:, :].astype(BF), wq_j,
                     preferred_element_type=jnp.float32).astype(BF)
        for i, ob in enumerate(_ORDER):
            qp[64 * i:64 * i + 64, :] = qf[64 * ob:64 * ob + 64, :]
        for (q0, qn), (a0, an), breg in CLS:
            qh = qp[q0:q0 + qn, :]
            k_a = k_buf[slot, a0:a0 + an, :].astype(BF)
            v_a = v_buf[slot, a0:a0 + an, :].astype(BF)
            s_a = lax.dot_general(
                qh, k_a, (((1,), (1,)), ((), ())),
                preferred_element_type=jnp.float32) * SCALE
            m_a = jnp.max(s_a, axis=-1, keepdims=True)
            if breg is None:
                w = jnp.exp(s_a - m_a)
                ctx = jnp.dot(w.astype(BF), v_a,
                              preferred_element_type=jnp.float32)
                ctx = ctx / jnp.sum(w, axis=-1, keepdims=True)
            else:
                b0, bn = breg
                s_b = lax.dot_general(
                    qh, k_buf[slot, b0:b0 + bn, :].astype(BF),
                    (((1,), (1,)), ((), ())),
                    preferred_element_type=jnp.float32) * SCALE
                rb = lax.broadcasted_iota(jnp.int32, (qn, 1), 0) // 64
                cb = lax.broadcasted_iota(jnp.int32, (1, bn), 1) // 64
                s_b = jnp.where(rb == cb, s_b, NEG)
                s_c = lax.dot_general(
                    qh, k_buf[slot, 0:64, :].astype(BF),
                    (((1,), (1,)), ((), ())),
                    preferred_element_type=jnp.float32) * SCALE
                m = jnp.maximum(
                    m_a,
                    jnp.maximum(jnp.max(s_b, axis=-1, keepdims=True),
                                jnp.max(s_c, axis=-1, keepdims=True)))
                e_a = jnp.exp(s_a - m)
                e_b = jnp.exp(s_b - m)
                e_c = jnp.exp(s_c - m)
                denom = (jnp.sum(e_a, axis=-1, keepdims=True)
                         + jnp.sum(e_b, axis=-1, keepdims=True)
                         + jnp.sum(e_c, axis=-1, keepdims=True))
                ctx = (jnp.dot(e_a.astype(BF), v_a,
                               preferred_element_type=jnp.float32)
                       + jnp.dot(e_b.astype(BF),
                                 v_buf[slot, b0:b0 + bn, :].astype(BF),
                                 preferred_element_type=jnp.float32)
                       + jnp.dot(e_c.astype(BF),
                                 v_buf[slot, 0:64, :].astype(BF),
                                 preferred_element_type=jnp.float32))
                ctx = ctx / denom
            part = jnp.dot(ctx.astype(BF), wo_j,
                           preferred_element_type=jnp.float32)
            if j == 0:
                out_ref[q0:q0 + qn, :] = part
            else:
                out_ref[q0:q0 + qn, :] = out_ref[q0:q0 + qn, :] + part

    def rs_desc(h, s):
        return pltpu.make_async_remote_copy(
            src_ref=rs_sb.at[h, pl.ds(s * SEGR, SEGR), :],
            dst_ref=rs_recv.at[h, pl.ds(s * SEGR, SEGR), :],
            send_sem=rs_ssem.at[h, s],
            recv_sem=rs_rsem.at[h, s],
            device_id=(right,),
            device_id_type=pl.DeviceIdType.MESH,
        )

    rs_sends = []
    rs_sb[0] = out_ref[pl.ds(my * CHUNK, CHUNK), :].astype(BF)
    for s in range(SEG):
        d = rs_desc(0, s)
        d.start()
        rs_sends.append(d)
    for h in range(N_DEV - 1):
        recv_c = (my - h - 1) % N_DEV
        for s in range(SEG):
            rs_desc(h, s).wait_recv()
            rows = pl.ds(recv_c * CHUNK + s * SEGR, SEGR)
            acc = (out_ref[rows, :]
                   + rs_recv[h, s * SEGR:(s + 1) * SEGR, :].astype(
                       jnp.float32))
            out_ref[rows, :] = acc
            if h < N_DEV - 2:
                rs_sb[h + 1, s * SEGR:(s + 1) * SEGR, :] = acc.astype(BF)
                d = rs_desc(h + 1, s)
                d.start()
                rs_sends.append(d)
    for d in rs_sends:
        d.wait_send()

    def ag_desc(h, s):
        if h == 0:
            src = ag_sb.at[pl.ds(s * SEGR, SEGR), :]
        else:
            src = ag_recv.at[h - 1, pl.ds(s * SEGR, SEGR), :]
        return pltpu.make_async_remote_copy(
            src_ref=src,
            dst_ref=ag_recv.at[h, pl.ds(s * SEGR, SEGR), :],
            send_sem=ag_ssem.at[h, s],
            recv_sem=ag_rsem.at[h, s],
            device_id=(right,),
            device_id_type=pl.DeviceIdType.MESH,
        )

    ag_sends = []
    ag_sb[...] = out_ref[
        pl.ds(((my + 1) % N_DEV) * CHUNK, CHUNK), :].astype(BF)
    for s in range(SEG):
        d = ag_desc(0, s)
        d.start()
        ag_sends.append(d)
    for h in range(N_DEV - 1):
        for s in range(SEG):
            ag_desc(h, s).wait_recv()
            if h < N_DEV - 2:
                d = ag_desc(h + 1, s)
                d.start()
                ag_sends.append(d)
            out_ref[pl.ds(((my - h) % N_DEV) * CHUNK + s * SEGR, SEGR), :] = (
                ag_recv[h, s * SEGR:(s + 1) * SEGR, :].astype(jnp.float32))
    for d in ag_sends:
        d.wait_send()

    for cyc in _CYCLES:
        k = len(cyc)
        blk_tmp[...] = out_ref[64 * cyc[k - 1]:64 * cyc[k - 1] + 64, :]
        for m in range(k - 1, 0, -1):
            out_ref[64 * cyc[m]:64 * cyc[m] + 64, :] = (
                out_ref[64 * cyc[m - 1]:64 * cyc[m - 1] + 64, :])
        out_ref[64 * cyc[0]:64 * cyc[0] + 64, :] = blk_tmp[...]


def kernel(x, Wq, K_ext, V_ext, Wo):
    x2 = x.reshape(SQ, D_MODEL)

    out = pl.pallas_call(
        _body,
        out_shape=jax.ShapeDtypeStruct((SQ, D_MODEL), jnp.float32),
        in_specs=[
            pl.BlockSpec(memory_space=pltpu.VMEM),
            pl.BlockSpec(memory_space=pltpu.VMEM),
            pl.BlockSpec(memory_space=pltpu.MemorySpace.HBM),
            pl.BlockSpec(memory_space=pltpu.MemorySpace.HBM),
            pl.BlockSpec(memory_space=pltpu.VMEM),
        ],
        out_specs=pl.BlockSpec(memory_space=pltpu.VMEM),
        scratch_shapes=[
            pltpu.VMEM((2, SKV, DH), jnp.float32),
            pltpu.VMEM((2, SKV, DH), jnp.float32),
            pltpu.VMEM((SQ, DH), BF),
            pltpu.VMEM((N_DEV - 1, CHUNK, D_MODEL), BF),
            pltpu.VMEM((N_DEV - 1, CHUNK, D_MODEL), BF),
            pltpu.VMEM((N_DEV - 1, CHUNK, D_MODEL), BF),
            pltpu.VMEM((CHUNK, D_MODEL), BF),
            pltpu.VMEM((64, D_MODEL), jnp.float32),
            pltpu.SemaphoreType.DMA((2, 2)),
            pltpu.SemaphoreType.DMA((N_DEV - 1, SEG)),
            pltpu.SemaphoreType.DMA((N_DEV - 1, SEG)),
            pltpu.SemaphoreType.DMA((N_DEV - 1, SEG)),
            pltpu.SemaphoreType.DMA((N_DEV - 1, SEG)),
        ],
        compiler_params=pltpu.CompilerParams(
            collective_id=0,
            vmem_limit_bytes=63 * 1024 * 1024,
        ),
    )(x2, Wq, K_ext, V_ext, Wo)
    return out.reshape(1, SQ, D_MODEL)
